# baseline (speedup 1.0000x reference)
import functools

import jax
import jax.numpy as jnp
from jax import lax
from jax.experimental import pallas as pl
from jax.experimental.pallas import tpu as pltpu
from jax.experimental.pallas import tpu_sc as plsc

_N = 10000
_E = 320000
_D = 128
_DE = 16
_H = 128

_NC = 2
_NS = 16
_NW = _NC * _NS

_PER_W = _E // _NW
_CHUNK = 400
_NCHUNK = _PER_W // _CHUNK
_BATCH = 80
_KB = _CHUNK // _BATCH
_ACC_PAD = 10240
_ZTILE = _ACC_PAD // _NS
_NBATCH = _PER_W // _BATCH

_BN = 2000


def _silu(x):
    return x / (1.0 + jnp.exp(-x))


def _premix_body(h_ref, wa_ref, wb_ref, t_ref):
    hb = h_ref[...]
    t_ref[0] = jnp.dot(
        hb, wa_ref[...], preferred_element_type=jnp.float32
    ).astype(jnp.bfloat16)
    t_ref[1] = jnp.dot(
        hb, wb_ref[...], preferred_element_type=jnp.float32
    ).astype(jnp.bfloat16)


def _premix(h, wa, wb):
    return pl.pallas_call(
        _premix_body,
        grid=(_N // _BN,),
        in_specs=[
            pl.BlockSpec((_BN, _D), lambda n: (n, 0)),
            pl.BlockSpec((_D, _H), lambda n: (0, 0)),
            pl.BlockSpec((_D, _H), lambda n: (0, 0)),
        ],
        out_specs=pl.BlockSpec((2, _BN, _H), lambda n: (0, n, 0)),
        out_shape=jax.ShapeDtypeStruct((2, _N, _H), jnp.bfloat16),
    )(h, wa, wb)


_sc_mesh = plsc.VectorSubcoreMesh(
    core_axis_name="c", subcore_axis_name="s", num_cores=_NC, num_subcores=_NS
)


_HW = _H // 2


@functools.partial(
    pl.kernel,
    out_type=jax.ShapeDtypeStruct((2, _E, _HW), jnp.int32),
    mesh=_sc_mesh,
    compiler_params=pltpu.CompilerParams(use_tc_tiling_on_sc=False),
    scratch_types=[
        pltpu.VMEM((_PER_W,), jnp.int32),
        pltpu.VMEM((_PER_W,), jnp.int32),
        pltpu.VMEM((_CHUNK, _HW), jnp.int32),
        pltpu.VMEM((_CHUNK, _HW), jnp.int32),
        pltpu.SemaphoreType.DMA,
        pltpu.SemaphoreType.DMA,
        pltpu.SemaphoreType.DMA,
        pltpu.SemaphoreType.DMA,
    ],
)
def _sc_gather(t0_hbm, t1_hbm, row_hbm, col_hbm, g_hbm,
               idx_a, idx_b, rows_a, rows_b, sga, sgb, ssa, ssb):
    c = lax.axis_index("c")
    s = lax.axis_index("s")
    wid = c * _NS + s
    base = wid * _PER_W
    pltpu.sync_copy(row_hbm.at[pl.ds(base, _PER_W)], idx_a)
    pltpu.sync_copy(col_hbm.at[pl.ds(base, _PER_W)], idx_b)

    def store(half, rows_v, sem, ci):
        return pltpu.make_async_copy(
            rows_v, g_hbm.at[half, pl.ds(base + ci * _CHUNK, _CHUNK)], sem
        )

    def fire(idx_v, rows_v, tab, sem, ci):
        return [
            pltpu.async_copy(
                tab.at[idx_v.at[pl.ds(ci * _CHUNK + j * _BATCH, _BATCH)]],
                rows_v.at[pl.ds(j * _BATCH, _BATCH)],
                sem,
            )
            for j in range(_KB)
        ]

    def chunk_body(ci, carry):
        @pl.when(ci > 0)
        def _():
            store(0, rows_a, ssa, ci - 1).wait()
            store(1, rows_b, ssb, ci - 1).wait()

        cps_a = fire(idx_a, rows_a, t0_hbm, sga, ci)
        cps_b = fire(idx_b, rows_b, t1_hbm, sgb, ci)
        for cp in cps_a:
            cp.wait()
        store(0, rows_a, ssa, ci).start()
        for cp in cps_b:
            cp.wait()
        store(1, rows_b, ssb, ci).start()
        return carry

    lax.fori_loop(0, _NCHUNK, chunk_body, 0)
    store(0, rows_a, ssa, _NCHUNK - 1).wait()
    store(1, rows_b, ssb, _NCHUNK - 1).wait()


def _edge_body(g_ref, attr_ref, w1c_ref, b1_ref, w2_ref, b2_ref, f_ref):
    z = (
        g_ref[0].astype(jnp.float32)
        + g_ref[1].astype(jnp.float32)
        + jnp.dot(attr_ref[...], w1c_ref[...], preferred_element_type=jnp.float32)
        + b1_ref[...]
    )
    z = _silu(z)
    f = jnp.dot(z, w2_ref[...], preferred_element_type=jnp.float32) + b2_ref[...]
    f_ref[...] = _silu(f)


def _edge_mlp(g, attr, w1c, b1, w2, b2):
    return pl.pallas_call(
        _edge_body,
        grid=(_E // _BN,),
        in_specs=[
            pl.BlockSpec((2, _BN, _H), lambda n: (0, n, 0)),
            pl.BlockSpec((_BN, _DE), lambda n: (n, 0)),
            pl.BlockSpec((_DE, _H), lambda n: (0, 0)),
            pl.BlockSpec((1, _H), lambda n: (0, 0)),
            pl.BlockSpec((_H, _H), lambda n: (0, 0)),
            pl.BlockSpec((1, _H), lambda n: (0, 0)),
        ],
        out_specs=pl.BlockSpec((_BN, _H), lambda n: (n, 0)),
        out_shape=jax.ShapeDtypeStruct((_E, _H), jnp.float32),
    )(g, attr, w1c, b1, w2, b2)


@functools.partial(
    pl.kernel,
    out_type=jax.ShapeDtypeStruct((_NC, _N, _H), jnp.float32),
    mesh=_sc_mesh,
    scratch_types=[
        pltpu.VMEM((_BATCH,), jnp.int32),
        pltpu.VMEM((_BATCH,), jnp.int32),
        pltpu.VMEM((_BATCH, _H), jnp.float32),
        pltpu.VMEM((_BATCH, _H), jnp.float32),
        pltpu.VMEM_SHARED((_ACC_PAD, _H), jnp.float32),
        pltpu.SemaphoreType.DMA,
        pltpu.SemaphoreType.DMA,
        pltpu.SemaphoreType.DMA,
        pltpu.SemaphoreType.DMA,
    ],
)
def _sc_scatter(f_hbm, row_hbm, out_hbm, i0, i1, f0, f1, acc, si0, si1, sf0, sf1):
    c = lax.axis_index("c")
    s = lax.axis_index("s")
    wid = c * _NS + s
    base = wid * _PER_W

    def zero_row(i, carry):
        for j in range(_H // 16):
            f0[i, pl.ds(j * 16, 16)] = jnp.zeros((16,), jnp.float32)
        return carry

    lax.fori_loop(0, _BATCH, zero_row, 0)
    for t in range(_ZTILE // _BATCH):
        pltpu.sync_copy(f0, acc.at[pl.ds(s * _ZTILE + t * _BATCH, _BATCH)])
    plsc.subcore_barrier()

    def start(e, iv, fvv, sem_i, sem_f):
        eb = base + e * _BATCH
        pltpu.async_copy(row_hbm.at[pl.ds(eb, _BATCH)], iv, sem_i)
        pltpu.async_copy(f_hbm.at[pl.ds(eb, _BATCH)], fvv, sem_f)

    def drain(e, iv, fvv, sem_i, sem_f):
        eb = base + e * _BATCH
        pltpu.make_async_copy(row_hbm.at[pl.ds(eb, _BATCH)], iv, sem_i).wait()
        pltpu.make_async_copy(f_hbm.at[pl.ds(eb, _BATCH)], fvv, sem_f).wait()

    start(0, i0, f0, si0, sf0)
    start(1, i1, f1, si1, sf1)

    def body(i, carry):
        e = 2 * i
        drain(e, i0, f0, si0, sf0)
        pltpu.sync_copy(f0, acc.at[i0], add=True)
        start(e + 2, i0, f0, si0, sf0)
        drain(e + 1, i1, f1, si1, sf1)
        pltpu.sync_copy(f1, acc.at[i1], add=True)

        @pl.when(i < _NBATCH // 2 - 1)
        def _():
            start(e + 3, i1, f1, si1, sf1)

        return carry

    lax.fori_loop(0, _NBATCH // 2, body, 0)
    drain(_NBATCH - 1, i0, f0, si0, sf0)
    pltpu.sync_copy(f0, acc.at[i0], add=True)
    plsc.subcore_barrier()
    @pl.when(s < _NS - 1)
    def _():
        pltpu.sync_copy(
            acc.at[pl.ds(s * _ZTILE, _ZTILE)],
            out_hbm.at[c, pl.ds(s * _ZTILE, _ZTILE)],
        )

    @pl.when(s == _NS - 1)
    def _():
        pltpu.sync_copy(
            acc.at[pl.ds((_NS - 1) * _ZTILE, _N - (_NS - 1) * _ZTILE)],
            out_hbm.at[c, pl.ds((_NS - 1) * _ZTILE, _N - (_NS - 1) * _ZTILE)],
        )


def _node_body(h_ref, p_ref, w1a_ref, w1b_ref, b1_ref, w2_ref, b2_ref, o_ref):
    hb = h_ref[...]
    agg = p_ref[0] + p_ref[1]
    z = _silu(
        jnp.dot(hb, w1a_ref[...], preferred_element_type=jnp.float32)
        + jnp.dot(agg, w1b_ref[...], preferred_element_type=jnp.float32)
        + b1_ref[...]
    )
    o_ref[...] = (
        jnp.dot(z, w2_ref[...], preferred_element_type=jnp.float32) + b2_ref[...] + hb
    )


def _node_mlp(h, parts, w1a, w1b, b1, w2, b2):
    return pl.pallas_call(
        _node_body,
        grid=(_N // _BN,),
        in_specs=[
            pl.BlockSpec((_BN, _D), lambda n: (n, 0)),
            pl.BlockSpec((2, _BN, _H), lambda n: (0, n, 0)),
            pl.BlockSpec((_D, _H), lambda n: (0, 0)),
            pl.BlockSpec((_H, _H), lambda n: (0, 0)),
            pl.BlockSpec((1, _H), lambda n: (0, 0)),
            pl.BlockSpec((_H, _D), lambda n: (0, 0)),
            pl.BlockSpec((1, _D), lambda n: (0, 0)),
        ],
        out_specs=pl.BlockSpec((_BN, _D), lambda n: (n, 0)),
        out_shape=jax.ShapeDtypeStruct((_N, _D), jnp.float32),
    )(h, parts, w1a, w1b, b1, w2, b2)


def kernel(h, edge_index, edge_attr, eW1, eb1, eW2, eb2, nW1, nb1, nW2, nb2):
    row = edge_index[0]
    col = edge_index[1]
    w1a = eW1[:_D]
    w1b = eW1[_D : 2 * _D]
    w1c = eW1[2 * _D :]

    t = _premix(h, w1a, w1b)
    t32 = jax.lax.bitcast_convert_type(
        t.reshape(2, _N, _HW, 2), jnp.int32
    )
    g32 = _sc_gather(t32[0], t32[1], row, col)
    g = jax.lax.bitcast_convert_type(g32, jnp.bfloat16).reshape(2, _E, _H)
    f = _edge_mlp(
        g, edge_attr, w1c, eb1.reshape(1, _H), eW2, eb2.reshape(1, _H)
    )
    parts = _sc_scatter(f, row)
    out = _node_mlp(
        h,
        parts,
        nW1[:_D],
        nW1[_D:],
        nb1.reshape(1, _H),
        nW2,
        nb2.reshape(1, _D),
    )
    return (out, edge_attr)


# speedup vs baseline: 2.7708x; 2.7708x over previous
import functools

import jax
import jax.numpy as jnp
from jax import lax
from jax.experimental import pallas as pl
from jax.experimental.pallas import tpu as pltpu
from jax.experimental.pallas import tpu_sc as plsc

_N = 10000
_E = 320000
_D = 128
_DE = 16
_H = 128

_NC = 2
_NS = 16
_NW = _NC * _NS

_EH = _E // 2
_PER_W = _EH // _NW
_BATCH = 40
_CHUNK = 200
_NCHUNK = _PER_W // _CHUNK
_KB = _CHUNK // _BATCH
_ACC_PAD = 10240
_ZTILE = _ACC_PAD // _NS
_NBATCH = _PER_W // _BATCH

_BN = 2000


def _silu(x):
    return x / (1.0 + jnp.exp(-x))


def _premix_body(h_ref, wa_ref, wb_ref, t_ref):
    hb = h_ref[...]
    t_ref[0] = jnp.dot(hb, wa_ref[...], preferred_element_type=jnp.float32)
    t_ref[1] = jnp.dot(hb, wb_ref[...], preferred_element_type=jnp.float32)


def _premix(h, wa, wb):
    return pl.pallas_call(
        _premix_body,
        grid=(_N // _BN,),
        in_specs=[
            pl.BlockSpec((_BN, _D), lambda n: (n, 0)),
            pl.BlockSpec((_D, _H), lambda n: (0, 0)),
            pl.BlockSpec((_D, _H), lambda n: (0, 0)),
        ],
        out_specs=pl.BlockSpec((2, _BN, _H), lambda n: (0, n, 0)),
        out_shape=jax.ShapeDtypeStruct((2, _N, _H), jnp.float32),
    )(h, wa, wb)


_sc_mesh = plsc.VectorSubcoreMesh(
    core_axis_name="c", subcore_axis_name="s", num_cores=_NC, num_subcores=_NS
)


@functools.partial(
    pl.kernel,
    out_type=jax.ShapeDtypeStruct((_EH, _H), jnp.float32),
    mesh=_sc_mesh,
    scratch_types=[
        pltpu.VMEM((_PER_W,), jnp.int32),
        pltpu.VMEM((_PER_W,), jnp.int32),
        pltpu.VMEM((_CHUNK, _H), jnp.float32),
        pltpu.VMEM((_CHUNK, _H), jnp.float32),
        pltpu.SemaphoreType.DMA,
        pltpu.SemaphoreType.DMA,
        pltpu.SemaphoreType.DMA,
        pltpu.SemaphoreType.DMA,
    ],
)
def _sc_gather(t0_hbm, t1_hbm, row_hbm, col_hbm, g_hbm,
               idx_a, idx_b, rows_a, rows_b, sga, sgb, ssa, ssb):
    c = lax.axis_index("c")
    s = lax.axis_index("s")
    wid = c * _NS + s
    base = wid * _PER_W
    pltpu.sync_copy(row_hbm.at[pl.ds(base, _PER_W)], idx_a)
    pltpu.sync_copy(col_hbm.at[pl.ds(base, _PER_W)], idx_b)

    def store(ci):
        return pltpu.make_async_copy(
            rows_a, g_hbm.at[pl.ds(base + ci * _CHUNK, _CHUNK)], ssa
        )

    def fire(idx_v, rows_v, tab, sem, ci):
        return [
            pltpu.async_copy(
                tab.at[idx_v.at[pl.ds(ci * _CHUNK + j * _BATCH, _BATCH)]],
                rows_v.at[pl.ds(j * _BATCH, _BATCH)],
                sem,
            )
            for j in range(_KB)
        ]

    def chunk_body(ci, carry):
        cps_b = fire(idx_b, rows_b, t1_hbm, sgb, ci)

        @pl.when(ci > 0)
        def _():
            store(ci - 1).wait()

        cps_a = fire(idx_a, rows_a, t0_hbm, sga, ci)
        for cp in cps_a + cps_b:
            cp.wait()

        def add_row(r, carry2):
            for j in range(_H // 16):
                sl = pl.ds(j * 16, 16)
                rows_a[r, sl] = rows_a[r, sl] + rows_b[r, sl]
            return carry2

        lax.fori_loop(0, _CHUNK, add_row, 0)
        store(ci).start()
        return carry

    lax.fori_loop(0, _NCHUNK, chunk_body, 0)
    store(_NCHUNK - 1).wait()


def _edge_body(g_ref, attr_ref, w1c_ref, b1_ref, w2_ref, b2_ref, f_ref):
    z = (
        g_ref[...]
        + jnp.dot(attr_ref[...], w1c_ref[...], preferred_element_type=jnp.float32)
        + b1_ref[...]
    )
    z = _silu(z)
    f = jnp.dot(z, w2_ref[...], preferred_element_type=jnp.float32) + b2_ref[...]
    f_ref[...] = _silu(f)


def _edge_mlp(g, attr, w1c, b1, w2, b2):
    return pl.pallas_call(
        _edge_body,
        grid=(_EH // _BN,),
        in_specs=[
            pl.BlockSpec((_BN, _H), lambda n: (n, 0)),
            pl.BlockSpec((_BN, _DE), lambda n: (n, 0)),
            pl.BlockSpec((_DE, _H), lambda n: (0, 0)),
            pl.BlockSpec((1, _H), lambda n: (0, 0)),
            pl.BlockSpec((_H, _H), lambda n: (0, 0)),
            pl.BlockSpec((1, _H), lambda n: (0, 0)),
        ],
        out_specs=pl.BlockSpec((_BN, _H), lambda n: (n, 0)),
        out_shape=jax.ShapeDtypeStruct((_EH, _H), jnp.float32),
    )(g, attr, w1c, b1, w2, b2)


@functools.partial(
    pl.kernel,
    out_type=jax.ShapeDtypeStruct((_NC, _N, _H), jnp.float32),
    mesh=_sc_mesh,
    scratch_types=[
        pltpu.VMEM((_BATCH,), jnp.int32),
        pltpu.VMEM((_BATCH,), jnp.int32),
        pltpu.VMEM((_BATCH, _H), jnp.float32),
        pltpu.VMEM((_BATCH, _H), jnp.float32),
        pltpu.VMEM_SHARED((_ACC_PAD, _H), jnp.float32),
        pltpu.SemaphoreType.DMA,
        pltpu.SemaphoreType.DMA,
        pltpu.SemaphoreType.DMA,
        pltpu.SemaphoreType.DMA,
    ],
)
def _sc_scatter(f_hbm, row_hbm, out_hbm, i0, i1, f0, f1, acc, si0, si1, sf0, sf1):
    c = lax.axis_index("c")
    s = lax.axis_index("s")
    wid = c * _NS + s
    base = wid * _PER_W

    def zero_row(i, carry):
        for j in range(_H // 16):
            f0[i, pl.ds(j * 16, 16)] = jnp.zeros((16,), jnp.float32)
        return carry

    lax.fori_loop(0, _BATCH, zero_row, 0)
    for t in range(_ZTILE // _BATCH):
        pltpu.sync_copy(f0, acc.at[pl.ds(s * _ZTILE + t * _BATCH, _BATCH)])
    plsc.subcore_barrier()

    def start(e, iv, fvv, sem_i, sem_f):
        eb = base + e * _BATCH
        pltpu.async_copy(row_hbm.at[pl.ds(eb, _BATCH)], iv, sem_i)
        pltpu.async_copy(f_hbm.at[pl.ds(eb, _BATCH)], fvv, sem_f)

    def drain(e, iv, fvv, sem_i, sem_f):
        eb = base + e * _BATCH
        pltpu.make_async_copy(row_hbm.at[pl.ds(eb, _BATCH)], iv, sem_i).wait()
        pltpu.make_async_copy(f_hbm.at[pl.ds(eb, _BATCH)], fvv, sem_f).wait()

    start(0, i0, f0, si0, sf0)
    start(1, i1, f1, si1, sf1)

    def body(i, carry):
        e = 2 * i
        drain(e, i0, f0, si0, sf0)
        pltpu.sync_copy(f0, acc.at[i0], add=True)
        start(e + 2, i0, f0, si0, sf0)
        drain(e + 1, i1, f1, si1, sf1)
        pltpu.sync_copy(f1, acc.at[i1], add=True)

        @pl.when(i < _NBATCH // 2 - 1)
        def _():
            start(e + 3, i1, f1, si1, sf1)

        return carry

    lax.fori_loop(0, _NBATCH // 2, body, 0)
    drain(_NBATCH - 1, i0, f0, si0, sf0)
    pltpu.sync_copy(f0, acc.at[i0], add=True)
    plsc.subcore_barrier()
    @pl.when(s < _NS - 1)
    def _():
        pltpu.sync_copy(
            acc.at[pl.ds(s * _ZTILE, _ZTILE)],
            out_hbm.at[c, pl.ds(s * _ZTILE, _ZTILE)],
        )

    @pl.when(s == _NS - 1)
    def _():
        pltpu.sync_copy(
            acc.at[pl.ds((_NS - 1) * _ZTILE, _N - (_NS - 1) * _ZTILE)],
            out_hbm.at[c, pl.ds((_NS - 1) * _ZTILE, _N - (_NS - 1) * _ZTILE)],
        )


def _node_body(h_ref, pa_ref, pb_ref, w1a_ref, w1b_ref, b1_ref, w2_ref, b2_ref, o_ref):
    hb = h_ref[...]
    agg = (pa_ref[0] + pa_ref[1]) + (pb_ref[0] + pb_ref[1])
    z = _silu(
        jnp.dot(hb, w1a_ref[...], preferred_element_type=jnp.float32)
        + jnp.dot(agg, w1b_ref[...], preferred_element_type=jnp.float32)
        + b1_ref[...]
    )
    o_ref[...] = (
        jnp.dot(z, w2_ref[...], preferred_element_type=jnp.float32) + b2_ref[...] + hb
    )


def _node_mlp(h, parts_a, parts_b, w1a, w1b, b1, w2, b2):
    return pl.pallas_call(
        _node_body,
        grid=(_N // _BN,),
        in_specs=[
            pl.BlockSpec((_BN, _D), lambda n: (n, 0)),
            pl.BlockSpec((2, _BN, _H), lambda n: (0, n, 0)),
            pl.BlockSpec((2, _BN, _H), lambda n: (0, n, 0)),
            pl.BlockSpec((_D, _H), lambda n: (0, 0)),
            pl.BlockSpec((_H, _H), lambda n: (0, 0)),
            pl.BlockSpec((1, _H), lambda n: (0, 0)),
            pl.BlockSpec((_H, _D), lambda n: (0, 0)),
            pl.BlockSpec((1, _D), lambda n: (0, 0)),
        ],
        out_specs=pl.BlockSpec((_BN, _D), lambda n: (n, 0)),
        out_shape=jax.ShapeDtypeStruct((_N, _D), jnp.float32),
    )(h, parts_a, parts_b, w1a, w1b, b1, w2, b2)


def kernel(h, edge_index, edge_attr, eW1, eb1, eW2, eb2, nW1, nb1, nW2, nb2):
    row = edge_index[0]
    col = edge_index[1]
    w1a = eW1[:_D]
    w1b = eW1[_D : 2 * _D]
    w1c = eW1[2 * _D :]

    t = _premix(h, w1a, w1b)
    b1 = eb1.reshape(1, _H)
    b2 = eb2.reshape(1, _H)
    parts = []
    for half in range(2):
        sl = slice(half * _EH, (half + 1) * _EH)
        g = _sc_gather(t[0], t[1], row[sl], col[sl])
        f = _edge_mlp(g, edge_attr[sl], w1c, b1, eW2, b2)
        parts.append(_sc_scatter(f, row[sl]))
    out = _node_mlp(
        h,
        parts[0],
        parts[1],
        nW1[:_D],
        nW1[_D:],
        nb1.reshape(1, _H),
        nW2,
        nb2.reshape(1, _D),
    )
    return (out, edge_attr)


# speedup vs baseline: 2.8190x; 1.0174x over previous
import functools

import jax
import jax.numpy as jnp
from jax import lax
from jax.experimental import pallas as pl
from jax.experimental.pallas import tpu as pltpu
from jax.experimental.pallas import tpu_sc as plsc

_N = 10000
_E = 320000
_D = 128
_DE = 16
_H = 128

_NC = 2
_NS = 16
_NW = _NC * _NS

_EH = _E // 2
_PER_W = _EH // _NW
_BATCH = 80
_CHUNK = 400
_NCHUNK = _PER_W // _CHUNK
_KB = _CHUNK // _BATCH
_GTAIL = _PER_W - _NCHUNK * _CHUNK
_TAILB = 40
_ACC_PAD = 10240
_ZTILE = _ACC_PAD // _NS
_NB80 = _PER_W // _BATCH

_BN = 2000


def _silu(x):
    return x / (1.0 + jnp.exp(-x))


def _premix_body(h_ref, wa_ref, wb_ref, t_ref):
    hb = h_ref[...]
    t_ref[0] = jnp.dot(hb, wa_ref[...], preferred_element_type=jnp.float32)
    t_ref[1] = jnp.dot(hb, wb_ref[...], preferred_element_type=jnp.float32)


def _premix(h, wa, wb):
    return pl.pallas_call(
        _premix_body,
        grid=(_N // _BN,),
        in_specs=[
            pl.BlockSpec((_BN, _D), lambda n: (n, 0)),
            pl.BlockSpec((_D, _H), lambda n: (0, 0)),
            pl.BlockSpec((_D, _H), lambda n: (0, 0)),
        ],
        out_specs=pl.BlockSpec((2, _BN, _H), lambda n: (0, n, 0)),
        out_shape=jax.ShapeDtypeStruct((2, _N, _H), jnp.float32),
    )(h, wa, wb)


_sc_mesh = plsc.VectorSubcoreMesh(
    core_axis_name="c", subcore_axis_name="s", num_cores=_NC, num_subcores=_NS
)


@functools.partial(
    pl.kernel,
    out_type=jax.ShapeDtypeStruct((_EH, _H), jnp.float32),
    mesh=_sc_mesh,
    scratch_types=[
        pltpu.VMEM((_PER_W,), jnp.int32),
        pltpu.VMEM((_PER_W,), jnp.int32),
        pltpu.VMEM((_CHUNK, _H), jnp.float32),
        pltpu.VMEM((_CHUNK, _H), jnp.float32),
        pltpu.SemaphoreType.DMA,
        pltpu.SemaphoreType.DMA,
        pltpu.SemaphoreType.DMA,
        pltpu.SemaphoreType.DMA,
    ],
)
def _sc_gather(t0_hbm, t1_hbm, row_hbm, col_hbm, g_hbm,
               idx_a, idx_b, rows_a, rows_b, sga, sgb, ssa, ssb):
    c = lax.axis_index("c")
    s = lax.axis_index("s")
    wid = c * _NS + s
    base = wid * _PER_W
    pltpu.sync_copy(row_hbm.at[pl.ds(base, _PER_W)], idx_a)
    pltpu.sync_copy(col_hbm.at[pl.ds(base, _PER_W)], idx_b)

    def store(ci):
        return pltpu.make_async_copy(
            rows_a, g_hbm.at[pl.ds(base + ci * _CHUNK, _CHUNK)], ssa
        )

    def fire(idx_v, rows_v, tab, sem, ci):
        return [
            pltpu.async_copy(
                tab.at[idx_v.at[pl.ds(ci * _CHUNK + j * _BATCH, _BATCH)]],
                rows_v.at[pl.ds(j * _BATCH, _BATCH)],
                sem,
            )
            for j in range(_KB)
        ]

    def chunk_body(ci, carry):
        cps_b = fire(idx_b, rows_b, t1_hbm, sgb, ci)

        @pl.when(ci > 0)
        def _():
            store(ci - 1).wait()

        cps_a = fire(idx_a, rows_a, t0_hbm, sga, ci)
        for cp in cps_a + cps_b:
            cp.wait()

        def add_row(r, carry2):
            for j in range(_H // 16):
                sl = pl.ds(j * 16, 16)
                rows_a[r, sl] = rows_a[r, sl] + rows_b[r, sl]
            return carry2

        lax.fori_loop(0, _CHUNK, add_row, 0)
        store(ci).start()
        return carry

    lax.fori_loop(0, _NCHUNK, chunk_body, 0)
    store(_NCHUNK - 1).wait()

    tbase = _NCHUNK * _CHUNK
    tail_sizes = (_BATCH, _BATCH, _GTAIL - 2 * _BATCH)
    cps = []
    for idx_v, rows_v, tab, sem in (
        (idx_a, rows_a, t0_hbm, sga),
        (idx_b, rows_b, t1_hbm, sgb),
    ):
        off = 0
        for bsz in tail_sizes:
            cps.append(
                pltpu.async_copy(
                    tab.at[idx_v.at[pl.ds(tbase + off, bsz)]],
                    rows_v.at[pl.ds(off, bsz)],
                    sem,
                )
            )
            off += bsz
    for cp in cps:
        cp.wait()

    def add_row_t(r, carry2):
        for j in range(_H // 16):
            sl = pl.ds(j * 16, 16)
            rows_a[r, sl] = rows_a[r, sl] + rows_b[r, sl]
        return carry2

    lax.fori_loop(0, _GTAIL, add_row_t, 0)
    pltpu.sync_copy(
        rows_a.at[pl.ds(0, _GTAIL)], g_hbm.at[pl.ds(base + tbase, _GTAIL)]
    )


def _edge_body(g_ref, attr_ref, w1c_ref, b1_ref, w2_ref, b2_ref, f_ref):
    z = (
        g_ref[...]
        + jnp.dot(attr_ref[...], w1c_ref[...], preferred_element_type=jnp.float32)
        + b1_ref[...]
    )
    z = _silu(z)
    f = jnp.dot(z, w2_ref[...], preferred_element_type=jnp.float32) + b2_ref[...]
    f_ref[...] = _silu(f)


def _edge_mlp(g, attr, w1c, b1, w2, b2):
    return pl.pallas_call(
        _edge_body,
        grid=(_EH // _BN,),
        in_specs=[
            pl.BlockSpec((_BN, _H), lambda n: (n, 0)),
            pl.BlockSpec((_BN, _DE), lambda n: (n, 0)),
            pl.BlockSpec((_DE, _H), lambda n: (0, 0)),
            pl.BlockSpec((1, _H), lambda n: (0, 0)),
            pl.BlockSpec((_H, _H), lambda n: (0, 0)),
            pl.BlockSpec((1, _H), lambda n: (0, 0)),
        ],
        out_specs=pl.BlockSpec((_BN, _H), lambda n: (n, 0)),
        out_shape=jax.ShapeDtypeStruct((_EH, _H), jnp.float32),
    )(g, attr, w1c, b1, w2, b2)


@functools.partial(
    pl.kernel,
    out_type=jax.ShapeDtypeStruct((_NC, _N, _H), jnp.float32),
    mesh=_sc_mesh,
    scratch_types=[
        pltpu.VMEM((_BATCH,), jnp.int32),
        pltpu.VMEM((_BATCH,), jnp.int32),
        pltpu.VMEM((_TAILB,), jnp.int32),
        pltpu.VMEM((_BATCH, _H), jnp.float32),
        pltpu.VMEM((_BATCH, _H), jnp.float32),
        pltpu.VMEM((_TAILB, _H), jnp.float32),
        pltpu.VMEM_SHARED((_ACC_PAD, _H), jnp.float32),
        pltpu.SemaphoreType.DMA,
        pltpu.SemaphoreType.DMA,
        pltpu.SemaphoreType.DMA,
        pltpu.SemaphoreType.DMA,
        pltpu.SemaphoreType.DMA,
        pltpu.SemaphoreType.DMA,
    ],
)
def _sc_scatter(f_hbm, row_hbm, out_hbm, i0, i1, it, f0, f1, ft, acc,
                si0, si1, sf0, sf1, sit, sft):
    c = lax.axis_index("c")
    s = lax.axis_index("s")
    wid = c * _NS + s
    base = wid * _PER_W

    def zero_row(i, carry):
        for j in range(_H // 16):
            f0[i, pl.ds(j * 16, 16)] = jnp.zeros((16,), jnp.float32)
        return carry

    lax.fori_loop(0, _BATCH, zero_row, 0)
    for t in range(_ZTILE // _BATCH):
        pltpu.sync_copy(f0, acc.at[pl.ds(s * _ZTILE + t * _BATCH, _BATCH)])
    plsc.subcore_barrier()

    def start(e, iv, fvv, sem_i, sem_f):
        eb = base + e * _BATCH
        pltpu.async_copy(row_hbm.at[pl.ds(eb, _BATCH)], iv, sem_i)
        pltpu.async_copy(f_hbm.at[pl.ds(eb, _BATCH)], fvv, sem_f)

    def drain(e, iv, fvv, sem_i, sem_f):
        eb = base + e * _BATCH
        pltpu.make_async_copy(row_hbm.at[pl.ds(eb, _BATCH)], iv, sem_i).wait()
        pltpu.make_async_copy(f_hbm.at[pl.ds(eb, _BATCH)], fvv, sem_f).wait()

    tb = base + _NB80 * _BATCH
    pltpu.async_copy(row_hbm.at[pl.ds(tb, _TAILB)], it, sit)
    pltpu.async_copy(f_hbm.at[pl.ds(tb, _TAILB)], ft, sft)
    start(0, i0, f0, si0, sf0)
    start(1, i1, f1, si1, sf1)

    def body(i, carry):
        e = 2 * i
        drain(e, i0, f0, si0, sf0)
        pltpu.sync_copy(f0, acc.at[i0], add=True)

        @pl.when(i < _NB80 // 2 - 1)
        def _():
            start(e + 2, i0, f0, si0, sf0)

        drain(e + 1, i1, f1, si1, sf1)
        pltpu.sync_copy(f1, acc.at[i1], add=True)

        @pl.when(i < _NB80 // 2 - 1)
        def _():
            start(e + 3, i1, f1, si1, sf1)

        return carry

    lax.fori_loop(0, _NB80 // 2, body, 0)
    pltpu.make_async_copy(row_hbm.at[pl.ds(tb, _TAILB)], it, sit).wait()
    pltpu.make_async_copy(f_hbm.at[pl.ds(tb, _TAILB)], ft, sft).wait()
    pltpu.sync_copy(ft, acc.at[it], add=True)
    plsc.subcore_barrier()
    @pl.when(s < _NS - 1)
    def _():
        pltpu.sync_copy(
            acc.at[pl.ds(s * _ZTILE, _ZTILE)],
            out_hbm.at[c, pl.ds(s * _ZTILE, _ZTILE)],
        )

    @pl.when(s == _NS - 1)
    def _():
        pltpu.sync_copy(
            acc.at[pl.ds((_NS - 1) * _ZTILE, _N - (_NS - 1) * _ZTILE)],
            out_hbm.at[c, pl.ds((_NS - 1) * _ZTILE, _N - (_NS - 1) * _ZTILE)],
        )


def _node_body(h_ref, pa_ref, pb_ref, w1a_ref, w1b_ref, b1_ref, w2_ref, b2_ref, o_ref):
    hb = h_ref[...]
    agg = (pa_ref[0] + pa_ref[1]) + (pb_ref[0] + pb_ref[1])
    z = _silu(
        jnp.dot(hb, w1a_ref[...], preferred_element_type=jnp.float32)
        + jnp.dot(agg, w1b_ref[...], preferred_element_type=jnp.float32)
        + b1_ref[...]
    )
    o_ref[...] = (
        jnp.dot(z, w2_ref[...], preferred_element_type=jnp.float32) + b2_ref[...] + hb
    )


def _node_mlp(h, parts_a, parts_b, w1a, w1b, b1, w2, b2):
    return pl.pallas_call(
        _node_body,
        grid=(_N // _BN,),
        in_specs=[
            pl.BlockSpec((_BN, _D), lambda n: (n, 0)),
            pl.BlockSpec((2, _BN, _H), lambda n: (0, n, 0)),
            pl.BlockSpec((2, _BN, _H), lambda n: (0, n, 0)),
            pl.BlockSpec((_D, _H), lambda n: (0, 0)),
            pl.BlockSpec((_H, _H), lambda n: (0, 0)),
            pl.BlockSpec((1, _H), lambda n: (0, 0)),
            pl.BlockSpec((_H, _D), lambda n: (0, 0)),
            pl.BlockSpec((1, _D), lambda n: (0, 0)),
        ],
        out_specs=pl.BlockSpec((_BN, _D), lambda n: (n, 0)),
        out_shape=jax.ShapeDtypeStruct((_N, _D), jnp.float32),
    )(h, parts_a, parts_b, w1a, w1b, b1, w2, b2)


def kernel(h, edge_index, edge_attr, eW1, eb1, eW2, eb2, nW1, nb1, nW2, nb2):
    row = edge_index[0]
    col = edge_index[1]
    w1a = eW1[:_D]
    w1b = eW1[_D : 2 * _D]
    w1c = eW1[2 * _D :]

    t = _premix(h, w1a, w1b)
    b1 = eb1.reshape(1, _H)
    b2 = eb2.reshape(1, _H)
    parts = []
    for half in range(2):
        sl = slice(half * _EH, (half + 1) * _EH)
        g = _sc_gather(t[0], t[1], row[sl], col[sl])
        f = _edge_mlp(g, edge_attr[sl], w1c, b1, eW2, b2)
        parts.append(_sc_scatter(f, row[sl]))
    out = _node_mlp(
        h,
        parts[0],
        parts[1],
        nW1[:_D],
        nW1[_D:],
        nb1.reshape(1, _H),
        nW2,
        nb2.reshape(1, _D),
    )
    return (out, edge_attr)


# speedup vs baseline: 2.8243x; 1.0019x over previous
import functools

import jax
import jax.numpy as jnp
from jax import lax
from jax.experimental import pallas as pl
from jax.experimental.pallas import tpu as pltpu
from jax.experimental.pallas import tpu_sc as plsc

_N = 10000
_E = 320000
_D = 128
_DE = 16
_H = 128

_NC = 2
_NS = 16
_NW = _NC * _NS

_EH = _E // 2
_PER_W = _EH // _NW
_BATCH = 80
_CHUNK = 200
_NCHUNK = _PER_W // _CHUNK
_CBATCH = (80, 80, 40)
_TAILB = 40
_ACC_PAD = 10240
_ZTILE = _ACC_PAD // _NS
_NB80 = _PER_W // _BATCH

_BN = 2000


def _silu(x):
    return x / (1.0 + jnp.exp(-x))


def _premix_body(h_ref, wa_ref, wb_ref, t_ref):
    hb = h_ref[...]
    t_ref[0] = jnp.dot(hb, wa_ref[...], preferred_element_type=jnp.float32)
    t_ref[1] = jnp.dot(hb, wb_ref[...], preferred_element_type=jnp.float32)


def _premix(h, wa, wb):
    return pl.pallas_call(
        _premix_body,
        grid=(_N // _BN,),
        in_specs=[
            pl.BlockSpec((_BN, _D), lambda n: (n, 0)),
            pl.BlockSpec((_D, _H), lambda n: (0, 0)),
            pl.BlockSpec((_D, _H), lambda n: (0, 0)),
        ],
        out_specs=pl.BlockSpec((2, _BN, _H), lambda n: (0, n, 0)),
        out_shape=jax.ShapeDtypeStruct((2, _N, _H), jnp.float32),
    )(h, wa, wb)


_sc_mesh = plsc.VectorSubcoreMesh(
    core_axis_name="c", subcore_axis_name="s", num_cores=_NC, num_subcores=_NS
)


@functools.partial(
    pl.kernel,
    out_type=jax.ShapeDtypeStruct((_EH, _H), jnp.float32),
    mesh=_sc_mesh,
    scratch_types=[
        pltpu.VMEM((_PER_W,), jnp.int32),
        pltpu.VMEM((_PER_W,), jnp.int32),
        pltpu.VMEM((_CHUNK, _H), jnp.float32),
        pltpu.VMEM((_CHUNK, _H), jnp.float32),
        pltpu.VMEM((_CHUNK, _H), jnp.float32),
        pltpu.VMEM((_CHUNK, _H), jnp.float32),
        pltpu.SemaphoreType.DMA,
        pltpu.SemaphoreType.DMA,
        pltpu.SemaphoreType.DMA,
        pltpu.SemaphoreType.DMA,
    ],
)
def _sc_gather(t0_hbm, t1_hbm, row_hbm, col_hbm, g_hbm,
               idx_a, idx_b, ra0, rb0, ra1, rb1, sg0, sg1, ss0, ss1):
    c = lax.axis_index("c")
    s = lax.axis_index("s")
    wid = c * _NS + s
    base = wid * _PER_W
    pltpu.sync_copy(row_hbm.at[pl.ds(base, _PER_W)], idx_a)
    pltpu.sync_copy(col_hbm.at[pl.ds(base, _PER_W)], idx_b)

    def store(ra, sem, ci):
        return pltpu.make_async_copy(
            ra, g_hbm.at[pl.ds(base + ci * _CHUNK, _CHUNK)], sem
        )

    def fire(ra, rb, sem, ci):
        cps = []
        for idx_v, rows_v, tab in ((idx_a, ra, t0_hbm), (idx_b, rb, t1_hbm)):
            off = 0
            for bsz in _CBATCH:
                cps.append(
                    pltpu.async_copy(
                        tab.at[idx_v.at[pl.ds(ci * _CHUNK + off, bsz)]],
                        rows_v.at[pl.ds(off, bsz)],
                        sem,
                    )
                )
                off += bsz
        return cps

    def add(ra, rb):
        def add_row(r, carry2):
            for j in range(_H // 16):
                sl = pl.ds(j * 16, 16)
                ra[r, sl] = ra[r, sl] + rb[r, sl]
            return carry2

        lax.fori_loop(0, _CHUNK, add_row, 0)

    def drain(cps):
        for cp in cps:
            cp.wait()

    drain(fire(ra0, rb0, sg0, 0))
    add(ra0, rb0)

    def body(i, carry):
        c0 = 2 * i

        @pl.when(i > 0)
        def _():
            store(ra1, ss1, c0 - 1).wait()

        cps1 = fire(ra1, rb1, sg1, c0 + 1)
        store(ra0, ss0, c0).start()
        drain(cps1)
        add(ra1, rb1)

        store(ra0, ss0, c0).wait()
        cps0 = fire(ra0, rb0, sg0, c0 + 2)
        store(ra1, ss1, c0 + 1).start()
        drain(cps0)
        add(ra0, rb0)
        return carry

    lax.fori_loop(0, (_NCHUNK - 1) // 2, body, 0)
    store(ra1, ss1, _NCHUNK - 2).wait()
    pltpu.sync_copy(ra0, g_hbm.at[pl.ds(base + (_NCHUNK - 1) * _CHUNK, _CHUNK)])


def _edge_body(g_ref, attr_ref, w1c_ref, b1_ref, w2_ref, b2_ref, f_ref):
    z = (
        g_ref[...]
        + jnp.dot(attr_ref[...], w1c_ref[...], preferred_element_type=jnp.float32)
        + b1_ref[...]
    )
    z = _silu(z)
    f = jnp.dot(z, w2_ref[...], preferred_element_type=jnp.float32) + b2_ref[...]
    f_ref[...] = _silu(f)


def _edge_mlp(g, attr, w1c, b1, w2, b2):
    return pl.pallas_call(
        _edge_body,
        grid=(_EH // _BN,),
        in_specs=[
            pl.BlockSpec((_BN, _H), lambda n: (n, 0)),
            pl.BlockSpec((_BN, _DE), lambda n: (n, 0)),
            pl.BlockSpec((_DE, _H), lambda n: (0, 0)),
            pl.BlockSpec((1, _H), lambda n: (0, 0)),
            pl.BlockSpec((_H, _H), lambda n: (0, 0)),
            pl.BlockSpec((1, _H), lambda n: (0, 0)),
        ],
        out_specs=pl.BlockSpec((_BN, _H), lambda n: (n, 0)),
        out_shape=jax.ShapeDtypeStruct((_EH, _H), jnp.float32),
    )(g, attr, w1c, b1, w2, b2)


@functools.partial(
    pl.kernel,
    out_type=jax.ShapeDtypeStruct((_NC, _N, _H), jnp.float32),
    mesh=_sc_mesh,
    scratch_types=[
        pltpu.VMEM((_BATCH,), jnp.int32),
        pltpu.VMEM((_BATCH,), jnp.int32),
        pltpu.VMEM((_TAILB,), jnp.int32),
        pltpu.VMEM((_BATCH, _H), jnp.float32),
        pltpu.VMEM((_BATCH, _H), jnp.float32),
        pltpu.VMEM((_TAILB, _H), jnp.float32),
        pltpu.VMEM_SHARED((_ACC_PAD, _H), jnp.float32),
        pltpu.SemaphoreType.DMA,
        pltpu.SemaphoreType.DMA,
        pltpu.SemaphoreType.DMA,
        pltpu.SemaphoreType.DMA,
        pltpu.SemaphoreType.DMA,
        pltpu.SemaphoreType.DMA,
    ],
)
def _sc_scatter(f_hbm, row_hbm, out_hbm, i0, i1, it, f0, f1, ft, acc,
                si0, si1, sf0, sf1, sit, sft):
    c = lax.axis_index("c")
    s = lax.axis_index("s")
    wid = c * _NS + s
    base = wid * _PER_W

    def zero_row(i, carry):
        for j in range(_H // 16):
            f0[i, pl.ds(j * 16, 16)] = jnp.zeros((16,), jnp.float32)
        return carry

    lax.fori_loop(0, _BATCH, zero_row, 0)
    for t in range(_ZTILE // _BATCH):
        pltpu.sync_copy(f0, acc.at[pl.ds(s * _ZTILE + t * _BATCH, _BATCH)])
    plsc.subcore_barrier()

    def start(e, iv, fvv, sem_i, sem_f):
        eb = base + e * _BATCH
        pltpu.async_copy(row_hbm.at[pl.ds(eb, _BATCH)], iv, sem_i)
        pltpu.async_copy(f_hbm.at[pl.ds(eb, _BATCH)], fvv, sem_f)

    def drain(e, iv, fvv, sem_i, sem_f):
        eb = base + e * _BATCH
        pltpu.make_async_copy(row_hbm.at[pl.ds(eb, _BATCH)], iv, sem_i).wait()
        pltpu.make_async_copy(f_hbm.at[pl.ds(eb, _BATCH)], fvv, sem_f).wait()

    tb = base + _NB80 * _BATCH
    pltpu.async_copy(row_hbm.at[pl.ds(tb, _TAILB)], it, sit)
    pltpu.async_copy(f_hbm.at[pl.ds(tb, _TAILB)], ft, sft)
    start(0, i0, f0, si0, sf0)
    start(1, i1, f1, si1, sf1)

    def body(i, carry):
        e = 2 * i
        drain(e, i0, f0, si0, sf0)
        pltpu.sync_copy(f0, acc.at[i0], add=True)

        @pl.when(i < _NB80 // 2 - 1)
        def _():
            start(e + 2, i0, f0, si0, sf0)

        drain(e + 1, i1, f1, si1, sf1)
        pltpu.sync_copy(f1, acc.at[i1], add=True)

        @pl.when(i < _NB80 // 2 - 1)
        def _():
            start(e + 3, i1, f1, si1, sf1)

        return carry

    lax.fori_loop(0, _NB80 // 2, body, 0)
    pltpu.make_async_copy(row_hbm.at[pl.ds(tb, _TAILB)], it, sit).wait()
    pltpu.make_async_copy(f_hbm.at[pl.ds(tb, _TAILB)], ft, sft).wait()
    pltpu.sync_copy(ft, acc.at[it], add=True)
    plsc.subcore_barrier()
    @pl.when(s < _NS - 1)
    def _():
        pltpu.sync_copy(
            acc.at[pl.ds(s * _ZTILE, _ZTILE)],
            out_hbm.at[c, pl.ds(s * _ZTILE, _ZTILE)],
        )

    @pl.when(s == _NS - 1)
    def _():
        pltpu.sync_copy(
            acc.at[pl.ds((_NS - 1) * _ZTILE, _N - (_NS - 1) * _ZTILE)],
            out_hbm.at[c, pl.ds((_NS - 1) * _ZTILE, _N - (_NS - 1) * _ZTILE)],
        )


def _node_body(h_ref, pa_ref, pb_ref, w1a_ref, w1b_ref, b1_ref, w2_ref, b2_ref, o_ref):
    hb = h_ref[...]
    agg = (pa_ref[0] + pa_ref[1]) + (pb_ref[0] + pb_ref[1])
    z = _silu(
        jnp.dot(hb, w1a_ref[...], preferred_element_type=jnp.float32)
        + jnp.dot(agg, w1b_ref[...], preferred_element_type=jnp.float32)
        + b1_ref[...]
    )
    o_ref[...] = (
        jnp.dot(z, w2_ref[...], preferred_element_type=jnp.float32) + b2_ref[...] + hb
    )


def _node_mlp(h, parts_a, parts_b, w1a, w1b, b1, w2, b2):
    return pl.pallas_call(
        _node_body,
        grid=(_N // _BN,),
        in_specs=[
            pl.BlockSpec((_BN, _D), lambda n: (n, 0)),
            pl.BlockSpec((2, _BN, _H), lambda n: (0, n, 0)),
            pl.BlockSpec((2, _BN, _H), lambda n: (0, n, 0)),
            pl.BlockSpec((_D, _H), lambda n: (0, 0)),
            pl.BlockSpec((_H, _H), lambda n: (0, 0)),
            pl.BlockSpec((1, _H), lambda n: (0, 0)),
            pl.BlockSpec((_H, _D), lambda n: (0, 0)),
            pl.BlockSpec((1, _D), lambda n: (0, 0)),
        ],
        out_specs=pl.BlockSpec((_BN, _D), lambda n: (n, 0)),
        out_shape=jax.ShapeDtypeStruct((_N, _D), jnp.float32),
    )(h, parts_a, parts_b, w1a, w1b, b1, w2, b2)


def kernel(h, edge_index, edge_attr, eW1, eb1, eW2, eb2, nW1, nb1, nW2, nb2):
    row = edge_index[0]
    col = edge_index[1]
    w1a = eW1[:_D]
    w1b = eW1[_D : 2 * _D]
    w1c = eW1[2 * _D :]

    t = _premix(h, w1a, w1b)
    b1 = eb1.reshape(1, _H)
    b2 = eb2.reshape(1, _H)
    parts = []
    for half in range(2):
        sl = slice(half * _EH, (half + 1) * _EH)
        g = _sc_gather(t[0], t[1], row[sl], col[sl])
        f = _edge_mlp(g, edge_attr[sl], w1c, b1, eW2, b2)
        parts.append(_sc_scatter(f, row[sl]))
    out = _node_mlp(
        h,
        parts[0],
        parts[1],
        nW1[:_D],
        nW1[_D:],
        nb1.reshape(1, _H),
        nW2,
        nb2.reshape(1, _D),
    )
    return (out, edge_attr)


# speedup vs baseline: 3.0659x; 1.0855x over previous
import functools

import jax
import jax.numpy as jnp
from jax import lax
from jax.experimental import pallas as pl
from jax.experimental.pallas import tpu as pltpu
from jax.experimental.pallas import tpu_sc as plsc

_N = 10000
_E = 320000
_D = 128
_DE = 16
_H = 128

_NC = 2
_NS = 16
_NW = _NC * _NS

_EH = _E // 2
_PER_W = _EH // _NW
_BATCH = 80
_CHUNK = 200
_NCHUNK = _PER_W // _CHUNK
_CBATCH = (80, 80, 40)
_TAILB = 40
_ACC_PAD = 10240
_ZTILE = _ACC_PAD // _NS
_NB80 = _PER_W // _BATCH

_BN = 2000
_BNE = 1280


def _silu(x):
    return x / (1.0 + jnp.exp(-x))


def _premix_body(h_ref, wa_ref, wb_ref, t_ref):
    hb = h_ref[...]
    t_ref[0] = jnp.dot(hb, wa_ref[...], preferred_element_type=jnp.float32)
    t_ref[1] = jnp.dot(hb, wb_ref[...], preferred_element_type=jnp.float32)


def _premix(h, wa, wb):
    return pl.pallas_call(
        _premix_body,
        grid=(_N // _BN,),
        in_specs=[
            pl.BlockSpec((_BN, _D), lambda n: (n, 0)),
            pl.BlockSpec((_D, _H), lambda n: (0, 0)),
            pl.BlockSpec((_D, _H), lambda n: (0, 0)),
        ],
        out_specs=pl.BlockSpec((2, _BN, _H), lambda n: (0, n, 0)),
        out_shape=jax.ShapeDtypeStruct((2, _N, _H), jnp.float32),
    )(h, wa, wb)


_sc_mesh = plsc.VectorSubcoreMesh(
    core_axis_name="c", subcore_axis_name="s", num_cores=_NC, num_subcores=_NS
)


def _make_gather(half):
    @functools.partial(
        pl.kernel,
        out_type=jax.ShapeDtypeStruct((_EH, _H), jnp.float32),
        mesh=_sc_mesh,
        scratch_types=[
            pltpu.VMEM((_PER_W,), jnp.int32),
            pltpu.VMEM((_PER_W,), jnp.int32),
            pltpu.VMEM((_CHUNK, _H), jnp.float32),
            pltpu.VMEM((_CHUNK, _H), jnp.float32),
            pltpu.VMEM((_CHUNK, _H), jnp.float32),
            pltpu.VMEM((_CHUNK, _H), jnp.float32),
            pltpu.SemaphoreType.DMA,
            pltpu.SemaphoreType.DMA,
            pltpu.SemaphoreType.DMA,
            pltpu.SemaphoreType.DMA,
        ],
    )
    def gather_k(t0_hbm, t1_hbm, ei_hbm, g_hbm,
                 idx_a, idx_b, ra0, rb0, ra1, rb1, sg0, sg1, ss0, ss1):
        c = lax.axis_index("c")
        s = lax.axis_index("s")
        wid = c * _NS + s
        base = wid * _PER_W
        pltpu.sync_copy(ei_hbm.at[pl.ds(half * _EH + base, _PER_W)], idx_a)
        pltpu.sync_copy(ei_hbm.at[pl.ds(_E + half * _EH + base, _PER_W)], idx_b)

        def store(ra, sem, ci):
            return pltpu.make_async_copy(
                ra, g_hbm.at[pl.ds(base + ci * _CHUNK, _CHUNK)], sem
            )

        def fire(ra, rb, sem, ci):
            cps = []
            for idx_v, rows_v, tab in ((idx_a, ra, t0_hbm), (idx_b, rb, t1_hbm)):
                off = 0
                for bsz in _CBATCH:
                    cps.append(
                        pltpu.async_copy(
                            tab.at[idx_v.at[pl.ds(ci * _CHUNK + off, bsz)]],
                            rows_v.at[pl.ds(off, bsz)],
                            sem,
                        )
                    )
                    off += bsz
            return cps

        def add(ra, rb):
            def add_row(r, carry2):
                for j in range(_H // 16):
                    sl = pl.ds(j * 16, 16)
                    ra[r, sl] = ra[r, sl] + rb[r, sl]
                return carry2

            lax.fori_loop(0, _CHUNK, add_row, 0)

        def drain(cps):
            for cp in cps:
                cp.wait()

        drain(fire(ra0, rb0, sg0, 0))
        add(ra0, rb0)

        def body(i, carry):
            c0 = 2 * i

            @pl.when(i > 0)
            def _():
                store(ra1, ss1, c0 - 1).wait()

            cps1 = fire(ra1, rb1, sg1, c0 + 1)
            store(ra0, ss0, c0).start()
            drain(cps1)
            add(ra1, rb1)

            store(ra0, ss0, c0).wait()
            cps0 = fire(ra0, rb0, sg0, c0 + 2)
            store(ra1, ss1, c0 + 1).start()
            drain(cps0)
            add(ra0, rb0)
            return carry

        lax.fori_loop(0, (_NCHUNK - 1) // 2, body, 0)
        store(ra1, ss1, _NCHUNK - 2).wait()
        pltpu.sync_copy(
            ra0, g_hbm.at[pl.ds(base + (_NCHUNK - 1) * _CHUNK, _CHUNK)]
        )

    return gather_k


_GATHER = tuple(_make_gather(h) for h in (0, 1))


def _edge_body(g_ref, attrt_ref, w1c_ref, b1_ref, w2_ref, b2_ref, f_ref):
    ea = lax.dot_general(
        attrt_ref[...],
        w1c_ref[...],
        (((0,), (0,)), ((), ())),
        preferred_element_type=jnp.float32,
    )
    z = _silu(g_ref[...] + ea + b1_ref[...])
    f = jnp.dot(z, w2_ref[...], preferred_element_type=jnp.float32) + b2_ref[...]
    f_ref[...] = _silu(f)


def _make_edge(half):
    hoff = half * (_EH // _BNE)
    return lambda g, attrt, w1c, b1, w2, b2: pl.pallas_call(
        _edge_body,
        grid=(_EH // _BNE,),
        in_specs=[
            pl.BlockSpec((_BNE, _H), lambda n: (n, 0)),
            pl.BlockSpec((_DE, _BNE), lambda n: (0, n + hoff)),
            pl.BlockSpec((_DE, _H), lambda n: (0, 0)),
            pl.BlockSpec((1, _H), lambda n: (0, 0)),
            pl.BlockSpec((_H, _H), lambda n: (0, 0)),
            pl.BlockSpec((1, _H), lambda n: (0, 0)),
        ],
        out_specs=pl.BlockSpec((_BNE, _H), lambda n: (n, 0)),
        out_shape=jax.ShapeDtypeStruct((_EH, _H), jnp.float32),
    )(g, attrt, w1c, b1, w2, b2)


_EDGE = tuple(_make_edge(h) for h in (0, 1))


def _make_scatter(half):
    @functools.partial(
        pl.kernel,
        out_type=jax.ShapeDtypeStruct((_NC, _N, _H), jnp.float32),
        mesh=_sc_mesh,
        scratch_types=[
            pltpu.VMEM((_BATCH,), jnp.int32),
            pltpu.VMEM((_BATCH,), jnp.int32),
            pltpu.VMEM((_TAILB,), jnp.int32),
            pltpu.VMEM((_BATCH, _H), jnp.float32),
            pltpu.VMEM((_BATCH, _H), jnp.float32),
            pltpu.VMEM((_TAILB, _H), jnp.float32),
            pltpu.VMEM_SHARED((_ACC_PAD, _H), jnp.float32),
            pltpu.SemaphoreType.DMA,
            pltpu.SemaphoreType.DMA,
            pltpu.SemaphoreType.DMA,
            pltpu.SemaphoreType.DMA,
            pltpu.SemaphoreType.DMA,
            pltpu.SemaphoreType.DMA,
        ],
    )
    def scatter_k(f_hbm, ei_hbm, out_hbm, i0, i1, it, f0, f1, ft, acc,
                  si0, si1, sf0, sf1, sit, sft):
        c = lax.axis_index("c")
        s = lax.axis_index("s")
        wid = c * _NS + s
        base = wid * _PER_W
        ibase = half * _EH + base

        def zero_row(i, carry):
            for j in range(_H // 16):
                f0[i, pl.ds(j * 16, 16)] = jnp.zeros((16,), jnp.float32)
            return carry

        lax.fori_loop(0, _BATCH, zero_row, 0)
        for t in range(_ZTILE // _BATCH):
            pltpu.sync_copy(f0, acc.at[pl.ds(s * _ZTILE + t * _BATCH, _BATCH)])
        plsc.subcore_barrier()

        def start(e, iv, fvv, sem_i, sem_f):
            eb = e * _BATCH
            pltpu.async_copy(ei_hbm.at[pl.ds(ibase + eb, _BATCH)], iv, sem_i)
            pltpu.async_copy(f_hbm.at[pl.ds(base + eb, _BATCH)], fvv, sem_f)

        def drain(e, iv, fvv, sem_i, sem_f):
            eb = e * _BATCH
            pltpu.make_async_copy(
                ei_hbm.at[pl.ds(ibase + eb, _BATCH)], iv, sem_i
            ).wait()
            pltpu.make_async_copy(
                f_hbm.at[pl.ds(base + eb, _BATCH)], fvv, sem_f
            ).wait()

        tb = _NB80 * _BATCH
        pltpu.async_copy(ei_hbm.at[pl.ds(ibase + tb, _TAILB)], it, sit)
        pltpu.async_copy(f_hbm.at[pl.ds(base + tb, _TAILB)], ft, sft)
        start(0, i0, f0, si0, sf0)
        start(1, i1, f1, si1, sf1)

        def body(i, carry):
            e = 2 * i
            drain(e, i0, f0, si0, sf0)
            pltpu.sync_copy(f0, acc.at[i0], add=True)

            @pl.when(i < _NB80 // 2 - 1)
            def _():
                start(e + 2, i0, f0, si0, sf0)

            drain(e + 1, i1, f1, si1, sf1)
            pltpu.sync_copy(f1, acc.at[i1], add=True)

            @pl.when(i < _NB80 // 2 - 1)
            def _():
                start(e + 3, i1, f1, si1, sf1)

            return carry

        lax.fori_loop(0, _NB80 // 2, body, 0)
        pltpu.make_async_copy(ei_hbm.at[pl.ds(ibase + tb, _TAILB)], it, sit).wait()
        pltpu.make_async_copy(f_hbm.at[pl.ds(base + tb, _TAILB)], ft, sft).wait()
        pltpu.sync_copy(ft, acc.at[it], add=True)
        plsc.subcore_barrier()
        @pl.when(s < _NS - 1)
        def _():
            pltpu.sync_copy(
                acc.at[pl.ds(s * _ZTILE, _ZTILE)],
                out_hbm.at[c, pl.ds(s * _ZTILE, _ZTILE)],
            )

        @pl.when(s == _NS - 1)
        def _():
            pltpu.sync_copy(
                acc.at[pl.ds((_NS - 1) * _ZTILE, _N - (_NS - 1) * _ZTILE)],
                out_hbm.at[c, pl.ds((_NS - 1) * _ZTILE, _N - (_NS - 1) * _ZTILE)],
            )

    return scatter_k


_SCATTER = tuple(_make_scatter(h) for h in (0, 1))


def _node_body(h_ref, pa_ref, pb_ref, w1a_ref, w1b_ref, b1_ref, w2_ref, b2_ref, o_ref):
    hb = h_ref[...]
    agg = (pa_ref[0] + pa_ref[1]) + (pb_ref[0] + pb_ref[1])
    z = _silu(
        jnp.dot(hb, w1a_ref[...], preferred_element_type=jnp.float32)
        + jnp.dot(agg, w1b_ref[...], preferred_element_type=jnp.float32)
        + b1_ref[...]
    )
    o_ref[...] = (
        jnp.dot(z, w2_ref[...], preferred_element_type=jnp.float32) + b2_ref[...] + hb
    )


def _node_mlp(h, parts_a, parts_b, w1a, w1b, b1, w2, b2):
    return pl.pallas_call(
        _node_body,
        grid=(_N // _BN,),
        in_specs=[
            pl.BlockSpec((_BN, _D), lambda n: (n, 0)),
            pl.BlockSpec((2, _BN, _H), lambda n: (0, n, 0)),
            pl.BlockSpec((2, _BN, _H), lambda n: (0, n, 0)),
            pl.BlockSpec((_D, _H), lambda n: (0, 0)),
            pl.BlockSpec((_H, _H), lambda n: (0, 0)),
            pl.BlockSpec((1, _H), lambda n: (0, 0)),
            pl.BlockSpec((_H, _D), lambda n: (0, 0)),
            pl.BlockSpec((1, _D), lambda n: (0, 0)),
        ],
        out_specs=pl.BlockSpec((_BN, _D), lambda n: (n, 0)),
        out_shape=jax.ShapeDtypeStruct((_N, _D), jnp.float32),
    )(h, parts_a, parts_b, w1a, w1b, b1, w2, b2)


def kernel(h, edge_index, edge_attr, eW1, eb1, eW2, eb2, nW1, nb1, nW2, nb2):
    ei = edge_index.reshape(2 * _E)
    attrt = edge_attr.T
    w1a = eW1[:_D]
    w1b = eW1[_D : 2 * _D]
    w1c = eW1[2 * _D :]

    t = _premix(h, w1a, w1b)
    b1 = eb1.reshape(1, _H)
    b2 = eb2.reshape(1, _H)
    parts = []
    for half in range(2):
        g = _GATHER[half](t[0], t[1], ei)
        f = _EDGE[half](g, attrt, w1c, b1, eW2, b2)
        parts.append(_SCATTER[half](f, ei))
    out = _node_mlp(
        h,
        parts[0],
        parts[1],
        nW1[:_D],
        nW1[_D:],
        nb1.reshape(1, _H),
        nW2,
        nb2.reshape(1, _D),
    )
    return (out, edge_attr)
